# baseline (speedup 1.0000x reference)
import functools

import jax
import jax.numpy as jnp
from jax import lax
from jax.experimental import pallas as pl
from jax.experimental.pallas import tpu as pltpu
from jax.experimental.pallas import tpu_sc as plsc

F32 = jnp.float32
I32 = jnp.int32

L = 16
NC, NS = 2, 16
HID = 128

E = 400000
EPT = 25024
EB = 4096
NBLK = (EPT + EB - 1) // EB
E_PAD = (NS - 1) * EPT + NBLK * EB

CH = 7936
K = 128
ZR = 128
RPT = CH // NS
GARB = CH

POOL_B = 128
BN = 512


def _mesh():
    return plsc.VectorSubcoreMesh(core_axis_name="c", subcore_axis_name="s",
                                  num_cores=NC, num_subcores=NS)


@functools.cache
def _make_segsum(n_pass):
    n_pad = n_pass * NC * CH

    @functools.partial(
        pl.kernel,
        out_type=[jax.ShapeDtypeStruct((n_pad, HID), F32),
                  jax.ShapeDtypeStruct((n_pad, L), F32)],
        mesh=_mesh(),
        compiler_params=pltpu.CompilerParams(needs_layout_passes=False),
        scratch_types=[
            pltpu.VMEM((EB,), I32),
            pltpu.VMEM((EB,), I32),
            pltpu.VMEM((K + 2 * L,), I32),
            pltpu.VMEM((K + 2 * L,), I32),
            pltpu.VMEM((1, K), I32),
            pltpu.VMEM((1, K), I32),
            pltpu.VMEM((K, HID), F32),
            pltpu.VMEM((ZR, L), F32),
            pltpu.VMEM((K, L), F32),
            pltpu.VMEM_SHARED((CH + 8, HID), F32),
            pltpu.VMEM_SHARED((CH + 8, L), F32),
            pltpu.SemaphoreType.DMA,
        ],
    )
    def seg(src_hbm, dst_hbm, x_hbm, sums_hbm, cnts_hbm,
            src_v, dst_v, sbuf, lbuf, sidx, lidx, rows_v, zcnt, ones_v,
            acc, cacc, sem):
        c = lax.axis_index("c")
        s = lax.axis_index("s")
        ebase = s * EPT

        zero = jnp.zeros((L,), F32)
        one = jnp.ones((L,), F32)

        def fill_zcnt(i, _):
            zcnt[i, :] = zero
            return 0
        lax.fori_loop(0, ZR, fill_zcnt, 0)

        def fill_ones(i, _):
            ones_v[i, :] = one
            return 0
        lax.fori_loop(0, K, fill_ones, 0)

        iota = lax.iota(I32, L)
        myrow = s * RPT

        def flush():
            def cp(j, _):
                sidx[0, pl.ds(j * L, L)] = sbuf[pl.ds(j * L, L)]
                lidx[0, pl.ds(j * L, L)] = lbuf[pl.ds(j * L, L)]
                return 0
            lax.fori_loop(0, K // L, cp, 0)
            pltpu.async_copy(x_hbm.at[sidx.at[0]], rows_v, sem).wait()
            pltpu.sync_copy(rows_v, acc.at[lidx.at[0]], add=True)
            pltpu.sync_copy(ones_v, cacc.at[lidx.at[0]], add=True)
            ts = sbuf[pl.ds(K, L)]
            tl = lbuf[pl.ds(K, L)]
            sbuf[pl.ds(0, L)] = ts
            lbuf[pl.ds(0, L)] = tl

        def one_pass(p, _):
            chunk = (p * NC) * CH + c * CH
            def zero_rows(i, _):
                for j in range(HID // L):
                    rows_v[i, pl.ds(j * L, L)] = zero
                return 0
            lax.fori_loop(0, K, zero_rows, 0)
            for j in range(RPT // ZR):
                pltpu.sync_copy(rows_v, acc.at[pl.ds(myrow + j * ZR, ZR)])
                pltpu.sync_copy(zcnt, cacc.at[pl.ds(myrow + j * ZR, ZR)])
            if RPT % ZR:
                base = myrow + (RPT // ZR) * ZR
                pltpu.sync_copy(rows_v.at[pl.ds(0, RPT % ZR)],
                                acc.at[pl.ds(base, RPT % ZR)])
                pltpu.sync_copy(zcnt.at[pl.ds(0, RPT % ZR)],
                                cacc.at[pl.ds(base, RPT % ZR)])
            plsc.subcore_barrier()

            def blk_body(b, cnt):
                bbase = b * EB
                pltpu.sync_copy(src_hbm.at[pl.ds(ebase + bbase, EB)], src_v)
                pltpu.sync_copy(dst_hbm.at[pl.ds(ebase + bbase, EB)], dst_v)

                def step(i, cnt):
                    sv = src_v[pl.ds(i * L, L)]
                    dv = dst_v[pl.ds(i * L, L)]
                    loc = dv - chunk
                    m = ((loc >= 0) & (loc < CH)
                         & ((ebase + bbase + i * L + iota) < E))
                    mi = m.astype(I32)
                    pos = cnt + jnp.cumsum(mi) - mi
                    idx = jnp.where(m, pos, K + L + iota)
                    plsc.store_scatter(sbuf, [idx], sv)
                    plsc.store_scatter(lbuf, [idx], loc)
                    cnt = cnt + jnp.sum(mi)

                    def do_flush(ct):
                        flush()
                        return ct - K
                    return lax.cond(cnt >= K, do_flush, lambda ct: ct, cnt)

                return lax.fori_loop(0, EB // L, step, cnt)

            cnt = lax.fori_loop(0, NBLK, blk_body, 0)

            def pad_tail(j, _):
                keep = (j * L + iota) < cnt
                sv = sbuf[pl.ds(j * L, L)]
                lv = lbuf[pl.ds(j * L, L)]
                sbuf[pl.ds(j * L, L)] = jnp.where(keep, sv, 0)
                lbuf[pl.ds(j * L, L)] = jnp.where(keep, lv, GARB)
                return 0
            lax.fori_loop(0, K // L, pad_tail, 0)
            flush()

            plsc.subcore_barrier()
            for j in range(RPT // ZR):
                pltpu.sync_copy(acc.at[pl.ds(myrow + j * ZR, ZR)],
                                sums_hbm.at[pl.ds(chunk + myrow + j * ZR, ZR)])
                pltpu.sync_copy(cacc.at[pl.ds(myrow + j * ZR, ZR)],
                                cnts_hbm.at[pl.ds(chunk + myrow + j * ZR, ZR)])
            if RPT % ZR:
                base = myrow + (RPT // ZR) * ZR
                pltpu.sync_copy(acc.at[pl.ds(base, RPT % ZR)],
                                sums_hbm.at[pl.ds(chunk + base, RPT % ZR)])
                pltpu.sync_copy(cacc.at[pl.ds(base, RPT % ZR)],
                                cnts_hbm.at[pl.ds(chunk + base, RPT % ZR)])
            plsc.subcore_barrier()
            return 0

        lax.fori_loop(0, n_pass, one_pass, 0)

    return seg


@functools.cache
def _make_pool(nblk_total):
    nbw = nblk_total // (NC * NS)

    @functools.partial(
        pl.kernel,
        out_type=[jax.ShapeDtypeStruct((NC, POOL_B, HID), F32),
                  jax.ShapeDtypeStruct((NC, POOL_B, L), F32)],
        mesh=_mesh(),
        compiler_params=pltpu.CompilerParams(needs_layout_passes=False),
        scratch_types=[
            pltpu.VMEM((128,), I32),
            pltpu.VMEM((128, HID), F32),
            pltpu.VMEM((POOL_B // NS, HID), F32),
            pltpu.VMEM((POOL_B // NS, L), F32),
            pltpu.VMEM((128, L), F32),
            pltpu.VMEM_SHARED((POOL_B, HID), F32),
            pltpu.VMEM_SHARED((POOL_B, L), F32),
        ],
    )
    def pool(x_hbm, ids_hbm, part_hbm, pcnt_hbm,
             ids_v, rows_v, zr, zc, ones_v, acc, cacc):
        c = lax.axis_index("c")
        s = lax.axis_index("s")
        w = c * NS + s
        zero = jnp.zeros((L,), F32)
        one = jnp.ones((L,), F32)
        rpt = POOL_B // NS
        for i in range(rpt):
            for j in range(HID // L):
                zr[i, pl.ds(j * L, L)] = zero
            zc[i, :] = zero

        def fill_ones(i, _):
            ones_v[i, :] = one
            return 0
        lax.fori_loop(0, 128, fill_ones, 0)

        pltpu.sync_copy(zr, acc.at[pl.ds(s * rpt, rpt)])
        pltpu.sync_copy(zc, cacc.at[pl.ds(s * rpt, rpt)])
        plsc.subcore_barrier()

        def body(b, _):
            blk = w * nbw + b
            pltpu.sync_copy(x_hbm.at[pl.ds(blk * 128, 128)], rows_v)
            pltpu.sync_copy(ids_hbm.at[pl.ds(blk * 128, 128)], ids_v)
            pltpu.sync_copy(rows_v, acc.at[ids_v], add=True)
            pltpu.sync_copy(ones_v, cacc.at[ids_v], add=True)
            return 0
        lax.fori_loop(0, nbw, body, 0)

        plsc.subcore_barrier()
        pltpu.sync_copy(acc.at[pl.ds(s * rpt, rpt)],
                        part_hbm.at[c].at[pl.ds(s * rpt, rpt)])
        pltpu.sync_copy(cacc.at[pl.ds(s * rpt, rpt)],
                        pcnt_hbm.at[c].at[pl.ds(s * rpt, rpt)])

    return pool


def _proj(x_pad, wT, b):
    npad = x_pad.shape[0]

    def body(x_ref, w_ref, b_ref, o_ref):
        o_ref[...] = jnp.maximum(
            jnp.dot(x_ref[...], w_ref[...], preferred_element_type=F32)
            + b_ref[...], 0.0)

    return pl.pallas_call(
        body, grid=(npad // BN,),
        in_specs=[pl.BlockSpec((BN, HID), lambda i: (i, 0)),
                  pl.BlockSpec((HID, HID), lambda i: (0, 0)),
                  pl.BlockSpec((1, HID), lambda i: (0, 0))],
        out_specs=pl.BlockSpec((BN, HID), lambda i: (i, 0)),
        out_shape=jax.ShapeDtypeStruct((npad, HID), F32),
    )(x_pad, wT, b)


def _combine1(s1, c1, x, wl1T, wrT, b):
    npad = x.shape[0]

    def body(s1_ref, c1_ref, x_ref, wl_ref, wr_ref, b_ref, o_ref):
        mean = s1_ref[...] / jnp.maximum(c1_ref[...][:, 0:1], 1.0)
        acc = jnp.dot(mean, wl_ref[...], preferred_element_type=F32)
        acc = acc + jnp.dot(x_ref[...], wr_ref[...], preferred_element_type=F32)
        o_ref[...] = jnp.maximum(acc + b_ref[...], 0.0)

    return pl.pallas_call(
        body, grid=(npad // BN,),
        in_specs=[pl.BlockSpec((BN, HID), lambda i: (i, 0)),
                  pl.BlockSpec((BN, L), lambda i: (i, 0)),
                  pl.BlockSpec((BN, HID), lambda i: (i, 0)),
                  pl.BlockSpec((HID, HID), lambda i: (0, 0)),
                  pl.BlockSpec((HID, HID), lambda i: (0, 0)),
                  pl.BlockSpec((1, HID), lambda i: (0, 0))],
        out_specs=pl.BlockSpec((BN, HID), lambda i: (i, 0)),
        out_shape=jax.ShapeDtypeStruct((npad, HID), F32),
    )(s1, c1, x, wl1T, wrT, b)


def _combine2(s1, c1, s2, c2, x, wl1T, wl2T, wrT, b):
    npad = x.shape[0]

    def body(s1_ref, c1_ref, s2_ref, c2_ref, x_ref,
             wl1_ref, wl2_ref, wr_ref, b_ref, o_ref):
        m1 = s1_ref[...] / jnp.maximum(c1_ref[...][:, 0:1], 1.0)
        m2 = s2_ref[...] / jnp.maximum(c2_ref[...][:, 0:1], 1.0)
        acc = jnp.dot(m1, wl1_ref[...], preferred_element_type=F32)
        acc = acc + jnp.dot(m2, wl2_ref[...], preferred_element_type=F32)
        acc = acc + jnp.dot(x_ref[...], wr_ref[...], preferred_element_type=F32)
        o_ref[...] = jnp.maximum(acc + b_ref[...], 0.0)

    return pl.pallas_call(
        body, grid=(npad // BN,),
        in_specs=[pl.BlockSpec((BN, HID), lambda i: (i, 0)),
                  pl.BlockSpec((BN, L), lambda i: (i, 0)),
                  pl.BlockSpec((BN, HID), lambda i: (i, 0)),
                  pl.BlockSpec((BN, L), lambda i: (i, 0)),
                  pl.BlockSpec((BN, HID), lambda i: (i, 0)),
                  pl.BlockSpec((HID, HID), lambda i: (0, 0)),
                  pl.BlockSpec((HID, HID), lambda i: (0, 0)),
                  pl.BlockSpec((HID, HID), lambda i: (0, 0)),
                  pl.BlockSpec((1, HID), lambda i: (0, 0))],
        out_specs=pl.BlockSpec((BN, HID), lambda i: (i, 0)),
        out_shape=jax.ShapeDtypeStruct((npad, HID), F32),
    )(s1, c1, s2, c2, x, wl1T, wl2T, wrT, b)


def _final(ph, pch, pv, pcv, pe, pce, scal_pad, whT, wvT, weT, wsT, b):
    def body(ph_r, pch_r, pv_r, pcv_r, pe_r, pce_r, sc_r,
             wh_r, wv_r, we_r, ws_r, b_r, o_r):
        def pooled(p_r, c_r):
            sm = p_r[0, 0:64, :] + p_r[1, 0:64, :]
            ct = c_r[0, 0:64, 0:1] + c_r[1, 0:64, 0:1]
            return sm / jnp.maximum(ct, 1.0)

        out = jnp.dot(pooled(ph_r[...], pch_r[...]), wh_r[...],
                      preferred_element_type=F32)
        out = out + jnp.dot(pooled(pv_r[...], pcv_r[...]), wv_r[...],
                            preferred_element_type=F32)
        out = out + jnp.dot(pooled(pe_r[...], pce_r[...]), we_r[...],
                            preferred_element_type=F32)
        out = out + jnp.dot(sc_r[...], ws_r[...], preferred_element_type=F32)
        o_r[...] = out + b_r[...]

    full = lambda shape: pl.BlockSpec(shape, lambda: tuple(0 for _ in shape))
    return pl.pallas_call(
        body,
        in_specs=[full((NC, POOL_B, HID)), full((NC, POOL_B, L)),
                  full((NC, POOL_B, HID)), full((NC, POOL_B, L)),
                  full((NC, POOL_B, HID)), full((NC, POOL_B, L)),
                  full((64, HID)),
                  full((HID, HID)), full((HID, HID)), full((HID, HID)),
                  full((HID, HID)), full((1, HID))],
        out_specs=full((64, HID)),
        out_shape=jax.ShapeDtypeStruct((64, HID), F32),
    )(ph, pch, pv, pcv, pe, pce, scal_pad, whT, wvT, weT, wsT, b)


NPAD = {"hex": 53248, "vertex": 102400, "edge": 151552}
NPASS = {"hex": 4, "vertex": 7, "edge": 10}


def kernel(x_hex, x_vertex, x_edge, ei_hv, ei_vh, ei_ve, ei_ev,
           batch_hex, batch_vertex, batch_edge, scalars, params):
    p = params

    def prep_x(x, np_):
        n, f = x.shape
        return jnp.pad(x.astype(F32), ((0, np_ - n), (0, HID - f)))

    def prep_w(w):
        return jnp.pad(w.astype(F32), ((0, 0), (0, HID - w.shape[1]))).T

    x = {
        "hex": _proj(prep_x(x_hex, NPAD["hex"]), prep_w(p["proj_hex_w"]),
                     p["proj_hex_b"].reshape(1, HID)),
        "vertex": _proj(prep_x(x_vertex, NPAD["vertex"]), prep_w(p["proj_vertex_w"]),
                        p["proj_vertex_b"].reshape(1, HID)),
        "edge": _proj(prep_x(x_edge, NPAD["edge"]), prep_w(p["proj_edge_w"]),
                      p["proj_edge_b"].reshape(1, HID)),
    }

    ei = {"hv": ei_hv, "vh": ei_vh, "ve": ei_ve, "ev": ei_ev}
    rels = [("hv", "hex", "vertex"), ("vh", "vertex", "hex"),
            ("ve", "vertex", "edge"), ("ev", "edge", "vertex")]

    epad = {n: (jnp.pad(e[0].astype(I32), (0, E_PAD - E)),
                jnp.pad(e[1].astype(I32), (0, E_PAD - E)))
            for n, e in ei.items()}

    for lp in p["layers"]:
        seg = {}
        for name, s_t, d_t in rels:
            src, dst = epad[name]
            n_pad = NPASS[d_t] * NC * CH
            msgs = jnp.take(x[s_t], src[:E], axis=0)
            ss = jax.ops.segment_sum(msgs, dst[:E], num_segments=n_pad)
            cc = jax.ops.segment_sum(jnp.ones((E,), F32), dst[:E],
                                     num_segments=n_pad)
            seg[name] = (ss, cc[:, None] * jnp.ones((1, L), F32))

        w = lambda name: lp[name]
        s1, c1 = seg["hv"]
        s2, c2 = seg["ev"]
        xv = _combine2(
            s1, c1, s2, c2, x["vertex"],
            w("hv")["Wl"].T * 0.5, w("ev")["Wl"].T * 0.5,
            (w("hv")["Wr"] + w("ev")["Wr"]).T * 0.5,
            ((w("hv")["bl"] + w("ev")["bl"]) * 0.5).reshape(1, HID))
        sh, ch = seg["vh"]
        xh = _combine1(sh, ch, x["hex"], w("vh")["Wl"].T, w("vh")["Wr"].T,
                       w("vh")["bl"].reshape(1, HID))
        se, ce = seg["ve"]
        xe = _combine1(se, ce, x["edge"], w("ve")["Wl"].T, w("ve")["Wr"].T,
                       w("ve")["bl"].reshape(1, HID))
        x = {"hex": xh, "vertex": xv, "edge": xe}

    def prep_ids(b, np_):
        n = b.shape[0]
        return jnp.pad(b.astype(I32), (0, np_ - n), constant_values=64)

    parts = {}
    for t, b in (("hex", batch_hex), ("vertex", batch_vertex),
                 ("edge", batch_edge)):
        ids = prep_ids(b, NPAD[t])
        ps = jax.ops.segment_sum(x[t], ids, num_segments=POOL_B)
        pc = jax.ops.segment_sum(jnp.ones((NPAD[t],), F32), ids,
                                 num_segments=POOL_B)
        parts[t] = (jnp.stack([ps, jnp.zeros_like(ps)]),
                    jnp.stack([pc[:, None] * jnp.ones((1, L), F32),
                               jnp.zeros((POOL_B, L), F32)]))

    fw = p["final_w"]
    n_sc = scalars.shape[1]
    wsT = jnp.pad(fw[:, 3 * HID:], ((0, 0), (0, HID - n_sc))).T
    scal_pad = jnp.pad(scalars.astype(F32), ((0, 0), (0, HID - n_sc)))
    return _final(parts["hex"][0], parts["hex"][1],
                  parts["vertex"][0], parts["vertex"][1],
                  parts["edge"][0], parts["edge"][1],
                  scal_pad,
                  fw[:, 0:HID].T, fw[:, HID:2 * HID].T, fw[:, 2 * HID:3 * HID].T,
                  wsT, p["final_b"].reshape(1, HID))
